# Initial kernel scaffold; baseline (speedup 1.0000x reference)
import jax
import jax.numpy as jnp
from jax.experimental import pallas as pl


def kernel(x, edge_index_l, edge_index_g, batch, W_l1, b_l1, W_l2, b_l2, W_g1, b_g1, W_g2, b_g2, W_fuse, b_fuse):
    raise NotImplementedError("write your pallas kernel here")


# speedup vs baseline: 3.5089x; 3.5089x over previous
import jax
import jax.numpy as jnp
from jax import lax
from jax.experimental import pallas as pl
from jax.experimental.pallas import tpu as pltpu
from jax.experimental.pallas import tpu_sc as plsc

N = 10000
E = 320000
G = 64
IN_D = 2063
HID = 256
OUT_D = 128

NC = 2
NS = 16
CH = 128

KPAD = 2176
NP = 10240
RT = 1024
NROW = NP // RT
RTC = 1000
NROWC = N // RTC
TRASH = N

AGG_TCH = 160
AGG_CHUNKS = AGG_TCH * NS
AGG_EPAD = AGG_CHUNKS * CH

RPT = NP // NS

_mesh = plsc.VectorSubcoreMesh(core_axis_name="c", subcore_axis_name="s",
                               num_cores=NC, num_subcores=NS)


def _agg_body(tab_hbm, src2_hbm, dst_hbm, out_hbm, src_v, dst_v, buf, acc, sem):
    c = lax.axis_index("c")
    s = lax.axis_index("s")
    pltpu.sync_copy(tab_hbm.at[pl.ds(c * NP + s * RPT, RPT)], acc.at[pl.ds(s * RPT, RPT)])
    plsc.subcore_barrier()

    @pl.loop(0, AGG_TCH // 8)
    def _grp(g):
        base = s * AGG_TCH + g * 8
        pltpu.sync_copy(src2_hbm.at[pl.ds(c * AGG_CHUNKS + base, 8)], src_v)
        pltpu.sync_copy(dst_hbm.at[pl.ds(base, 8)], dst_v)
        for k in range(8):
            pltpu.async_copy(tab_hbm.at[src_v.at[k]], buf, sem).wait()
            pltpu.sync_copy(buf, acc.at[dst_v.at[k]], add=True)

    plsc.subcore_barrier()
    pltpu.sync_copy(acc.at[pl.ds(s * RPT, RPT)], out_hbm.at[pl.ds(c * NP + s * RPT, RPT)])


_agg_kernel = pl.kernel(
    _agg_body, mesh=_mesh,
    out_type=jax.ShapeDtypeStruct((2 * NP, 128), jnp.float32),
    scratch_types=[
        pltpu.VMEM((8, CH), jnp.int32),
        pltpu.VMEM((8, CH), jnp.int32),
        pltpu.VMEM((CH, 128), jnp.float32),
        pltpu.VMEM_SHARED((NP, 128), jnp.float32),
        pltpu.SemaphoreType.DMA,
    ],
)


def _mm_body(x_ref, w_ref, degl_ref, degg_ref, hp_ref, dinv_ref):
    j = pl.program_id(1)
    h = jnp.dot(x_ref[...], w_ref[...], preferred_element_type=jnp.float32)
    dinv_l = lax.rsqrt(jnp.maximum(degl_ref[:, 0:1], 1.0))
    dinv_g = lax.rsqrt(jnp.maximum(degg_ref[:, 0:1], 1.0))
    dinv = jnp.where(j < 2, dinv_l, dinv_g)
    hp_ref[...] = (h * dinv)[None]
    dinv_ref[...] = jnp.concatenate([dinv_l, dinv_g], axis=1)


def _matmul_scale(x_p, w_cat, deg_l, deg_g):
    return pl.pallas_call(
        _mm_body,
        grid=(NROW, 4),
        in_specs=[
            pl.BlockSpec((RT, KPAD), lambda i, j: (i, 0)),
            pl.BlockSpec((KPAD, 128), lambda i, j: (0, j)),
            pl.BlockSpec((RT, 128), lambda i, j: (i, 0)),
            pl.BlockSpec((RT, 128), lambda i, j: (i, 0)),
        ],
        out_specs=[
            pl.BlockSpec((1, RT, 128), lambda i, j: (j, i, 0)),
            pl.BlockSpec((RT, 2), lambda i, j: (i, 0)),
        ],
        out_shape=[
            jax.ShapeDtypeStruct((4, NP, 128), jnp.float32),
            jax.ShapeDtypeStruct((NP, 2), jnp.float32),
        ],
    )(x_p, w_cat, deg_l, deg_g)


def _mid_body(sl_ref, sg_ref, dinv_ref, bl_ref, bg_ref, ml_ref, mg_ref):
    dl = dinv_ref[:, 0:1][None]
    dg = dinv_ref[:, 1:2][None]
    ml_ref[...] = dl * jax.nn.relu(dl * sl_ref[...] + bl_ref[...][:, None, :])
    mg_ref[...] = dg * jax.nn.relu(dg * sg_ref[...] + bg_ref[...][:, None, :])


def _midmap(S_l, S_g, dinv2, b_l1, b_g1):
    return pl.pallas_call(
        _mid_body,
        grid=(NROW,),
        in_specs=[
            pl.BlockSpec((2, RT, 128), lambda i: (0, i, 0)),
            pl.BlockSpec((2, RT, 128), lambda i: (0, i, 0)),
            pl.BlockSpec((RT, 2), lambda i: (i, 0)),
            pl.BlockSpec((2, 128), lambda i: (0, 0)),
            pl.BlockSpec((2, 128), lambda i: (0, 0)),
        ],
        out_specs=[
            pl.BlockSpec((2, RT, 128), lambda i: (0, i, 0)),
            pl.BlockSpec((2, RT, 128), lambda i: (0, i, 0)),
        ],
        out_shape=[
            jax.ShapeDtypeStruct((2, NP, 128), jnp.float32),
            jax.ShapeDtypeStruct((2, NP, 128), jnp.float32),
        ],
    )(S_l, S_g, dinv2, b_l1, b_g1)


def _pool_body(s2l_ref, s2g_ref, dinv_ref, batch_ref, wl2_ref, wg2_ref, bl2_ref,
               bg2_ref, wf_ref, bf_ref, out_ref, pl_acc, pg_acc, cnt_acc):
    i = pl.program_id(0)
    batch_blk = batch_ref[0]
    gids = lax.broadcasted_iota(jnp.int32, (G, RTC), 0)
    mask = (gids == batch_blk).astype(jnp.float32)

    dl = dinv_ref[:, 0:1]
    dg = dinv_ref[:, 1:2]
    s2l = jnp.concatenate([s2l_ref[0], s2l_ref[1]], axis=1) * dl
    s2g = jnp.concatenate([s2g_ref[0], s2g_ref[1]], axis=1) * dg

    @pl.when(i == 0)
    def _():
        pl_acc[...] = jnp.zeros_like(pl_acc)
        pg_acc[...] = jnp.zeros_like(pg_acc)
        cnt_acc[...] = jnp.zeros_like(cnt_acc)

    pl_acc[...] += jnp.dot(mask, s2l, preferred_element_type=jnp.float32)
    pg_acc[...] += jnp.dot(mask, s2g, preferred_element_type=jnp.float32)
    cnt_acc[...] += jnp.sum(mask, axis=1, keepdims=True) * jnp.ones((G, 128), jnp.float32)

    @pl.when(i == NROWC - 1)
    def _():
        cnt = cnt_acc[:, 0:1]
        cmax = jnp.maximum(cnt, 1.0)
        nz = (cnt > 0.0).astype(jnp.float32)
        xl = jnp.dot(pl_acc[...] / cmax, wl2_ref[...],
                     preferred_element_type=jnp.float32) + bl2_ref[...] * nz
        xg = jnp.dot(pg_acc[...] / cmax, wg2_ref[...],
                     preferred_element_type=jnp.float32) + bg2_ref[...] * nz
        wf_top = wf_ref[0:128, :]
        wf_bot = wf_ref[128:256, :]
        fused = (jnp.dot(xl, wf_top, preferred_element_type=jnp.float32)
                 + jnp.dot(xg, wf_bot, preferred_element_type=jnp.float32)
                 + bf_ref[...])
        out_ref[...] = jax.nn.relu(fused)


def _pool_fuse(S2_l, S2_g, dinv2, batch3, W_l2, W_g2, b_l2, b_g2, W_fuse, b_fuse):
    return pl.pallas_call(
        _pool_body,
        grid=(NROWC,),
        in_specs=[
            pl.BlockSpec((2, RTC, 128), lambda i: (0, i, 0)),
            pl.BlockSpec((2, RTC, 128), lambda i: (0, i, 0)),
            pl.BlockSpec((RTC, 2), lambda i: (i, 0)),
            pl.BlockSpec((1, 1, RTC), lambda i: (i, 0, 0)),
            pl.BlockSpec((HID, 128), lambda i: (0, 0)),
            pl.BlockSpec((HID, 128), lambda i: (0, 0)),
            pl.BlockSpec((1, 128), lambda i: (0, 0)),
            pl.BlockSpec((1, 128), lambda i: (0, 0)),
            pl.BlockSpec((2 * OUT_D, 128), lambda i: (0, 0)),
            pl.BlockSpec((1, 128), lambda i: (0, 0)),
        ],
        out_specs=pl.BlockSpec((G, 128), lambda i: (0, 0)),
        out_shape=jax.ShapeDtypeStruct((G, OUT_D), jnp.float32),
        scratch_shapes=[
            pltpu.VMEM((G, HID), jnp.float32),
            pltpu.VMEM((G, HID), jnp.float32),
            pltpu.VMEM((G, 128), jnp.float32),
        ],
    )(S2_l, S2_g, dinv2, batch3, W_l2, W_g2, b_l2, b_g2, W_fuse, b_fuse)


def _prep_agg_idx(src, dst):
    srcp = jnp.concatenate([src, jnp.zeros((AGG_EPAD - E,), jnp.int32)])
    src2 = jnp.stack([srcp, srcp + NP]).reshape(NC * AGG_CHUNKS, CH)
    dstp = jnp.concatenate([dst, jnp.full((AGG_EPAD - E,), TRASH, jnp.int32)])
    return src2, dstp.reshape(AGG_CHUNKS, CH)


@jax.jit
def _run(x, edge_index_l, edge_index_g, batch, W_l1, b_l1, W_l2, b_l2,
         W_g1, b_g1, W_g2, b_g2, W_fuse, b_fuse):
    f32 = jnp.float32
    src2_l, dstp_l = _prep_agg_idx(edge_index_l[0], edge_index_l[1])
    src2_g, dstp_g = _prep_agg_idx(edge_index_g[0], edge_index_g[1])
    ones_tab = jnp.ones((2 * NP, 128), f32)
    deg_l = _agg_kernel(ones_tab, src2_l, dstp_l)
    deg_g = _agg_kernel(ones_tab, src2_g, dstp_g)

    x_p = jnp.pad(x, ((0, NP - N), (0, KPAD - IN_D)))
    w_cat = jnp.pad(jnp.concatenate([W_l1, W_g1], axis=1), ((0, KPAD - IN_D), (0, 0)))
    Hp, dinv2 = _matmul_scale(x_p, w_cat, deg_l, deg_g)

    S_l = _agg_kernel(Hp[0:2].reshape(2 * NP, 128), src2_l, dstp_l).reshape(2, NP, 128)
    S_g = _agg_kernel(Hp[2:4].reshape(2 * NP, 128), src2_g, dstp_g).reshape(2, NP, 128)

    Mp_l, Mp_g = _midmap(S_l, S_g, dinv2, b_l1.reshape(2, 128), b_g1.reshape(2, 128))

    S2_l = _agg_kernel(Mp_l.reshape(2 * NP, 128), src2_l, dstp_l).reshape(2, NP, 128)
    S2_g = _agg_kernel(Mp_g.reshape(2 * NP, 128), src2_g, dstp_g).reshape(2, NP, 128)

    batch3 = batch.reshape(NROWC, 1, RTC)
    out = _pool_fuse(S2_l, S2_g, dinv2, batch3, W_l2, W_g2,
                     b_l2.reshape(1, 128), b_g2.reshape(1, 128),
                     W_fuse, b_fuse.reshape(1, 128))
    return out


def kernel(x, edge_index_l, edge_index_g, batch, W_l1, b_l1, W_l2, b_l2,
           W_g1, b_g1, W_g2, b_g2, W_fuse, b_fuse):
    return _run(x, edge_index_l, edge_index_g, batch, W_l1, b_l1, W_l2, b_l2,
                W_g1, b_g1, W_g2, b_g2, W_fuse, b_fuse)


# speedup vs baseline: 4.2133x; 1.2008x over previous
import jax
import jax.numpy as jnp
from jax import lax
from jax.experimental import pallas as pl
from jax.experimental.pallas import tpu as pltpu
from jax.experimental.pallas import tpu_sc as plsc

N = 10000
E = 320000
G = 64
IN_D = 2063
HID = 256
OUT_D = 128

NC = 2
NS = 16
CH = 128

KPAD = 2176
NP = 10240
RT = 1024
NROW = NP // RT
RTC = 1000
NROWC = N // RTC
TRASH = N

AGG_TCH = 160
AGG_CHUNKS = AGG_TCH * NS
AGG_EPAD = AGG_CHUNKS * CH

RPT = NP // NS

_mesh = plsc.VectorSubcoreMesh(core_axis_name="c", subcore_axis_name="s",
                               num_cores=NC, num_subcores=NS)


NG = AGG_TCH // 8


def _agg_body(tab_hbm, src2_hbm, dst_hbm, out_hbm, src_v, dst_v, buf, acc,
              sem0, sem1):
    c = lax.axis_index("c")
    s = lax.axis_index("s")
    sems = (sem0, sem1)
    pltpu.sync_copy(tab_hbm.at[pl.ds(c * NP + s * RPT, RPT)], acc.at[pl.ds(s * RPT, RPT)])
    plsc.subcore_barrier()

    base = c * AGG_CHUNKS + s * AGG_TCH
    pltpu.sync_copy(src2_hbm.at[pl.ds(base, 8)], src_v.at[0])
    pltpu.sync_copy(dst_hbm.at[pl.ds(s * AGG_TCH, 8)], dst_v.at[0])
    pltpu.async_copy(tab_hbm.at[src_v.at[0].at[0]], buf.at[0], sem0)
    pltpu.async_copy(tab_hbm.at[src_v.at[0].at[1]], buf.at[1], sem1)

    @pl.loop(0, NG)
    def _grp(g):
        gp = g % 2
        gn = (g + 1) % 2

        @pl.when(g + 1 < NG)
        def _():
            pltpu.sync_copy(src2_hbm.at[pl.ds(base + (g + 1) * 8, 8)], src_v.at[gn])
            pltpu.sync_copy(dst_hbm.at[pl.ds(s * AGG_TCH + (g + 1) * 8, 8)], dst_v.at[gn])

        for k in range(8):
            p = k % 2
            pltpu.make_async_copy(tab_hbm.at[src_v.at[gp].at[k]], buf.at[p],
                                  sems[p]).wait()
            pltpu.sync_copy(buf.at[p], acc.at[dst_v.at[gp].at[k]], add=True)
            if k < 6:
                pltpu.async_copy(tab_hbm.at[src_v.at[gp].at[k + 2]], buf.at[p],
                                 sems[p])
            else:
                @pl.when(g + 1 < NG)
                def _():
                    pltpu.async_copy(tab_hbm.at[src_v.at[gn].at[k - 6]], buf.at[p],
                                     sems[p])

    plsc.subcore_barrier()
    pltpu.sync_copy(acc.at[pl.ds(s * RPT, RPT)], out_hbm.at[pl.ds(c * NP + s * RPT, RPT)])


_agg_kernel = pl.kernel(
    _agg_body, mesh=_mesh,
    out_type=jax.ShapeDtypeStruct((2 * NP, 128), jnp.float32),
    scratch_types=[
        pltpu.VMEM((2, 8, CH), jnp.int32),
        pltpu.VMEM((2, 8, CH), jnp.int32),
        pltpu.VMEM((2, CH, 128), jnp.float32),
        pltpu.VMEM_SHARED((NP, 128), jnp.float32),
        pltpu.SemaphoreType.DMA,
        pltpu.SemaphoreType.DMA,
    ],
)


def _mm_body(x_ref, w_ref, degl_ref, degg_ref, hp_ref, dinv_ref):
    j = pl.program_id(1)
    h = jnp.dot(x_ref[...], w_ref[...], preferred_element_type=jnp.float32)
    dinv_l = lax.rsqrt(jnp.maximum(degl_ref[:, 0:1], 1.0))
    dinv_g = lax.rsqrt(jnp.maximum(degg_ref[:, 0:1], 1.0))
    dinv = jnp.where(j < 2, dinv_l, dinv_g)
    hp_ref[...] = (h * dinv)[None]
    dinv_ref[...] = jnp.concatenate([dinv_l, dinv_g], axis=1)


def _matmul_scale(x_p, w_cat, deg_l, deg_g):
    return pl.pallas_call(
        _mm_body,
        grid=(NROW, 4),
        in_specs=[
            pl.BlockSpec((RT, KPAD), lambda i, j: (i, 0)),
            pl.BlockSpec((KPAD, 128), lambda i, j: (0, j)),
            pl.BlockSpec((RT, 128), lambda i, j: (i, 0)),
            pl.BlockSpec((RT, 128), lambda i, j: (i, 0)),
        ],
        out_specs=[
            pl.BlockSpec((1, RT, 128), lambda i, j: (j, i, 0)),
            pl.BlockSpec((RT, 2), lambda i, j: (i, 0)),
        ],
        out_shape=[
            jax.ShapeDtypeStruct((4, NP, 128), jnp.float32),
            jax.ShapeDtypeStruct((NP, 2), jnp.float32),
        ],
    )(x_p, w_cat, deg_l, deg_g)


def _mid_body(sl_ref, sg_ref, dinv_ref, bl_ref, bg_ref, ml_ref, mg_ref):
    dl = dinv_ref[:, 0:1][None]
    dg = dinv_ref[:, 1:2][None]
    ml_ref[...] = dl * jax.nn.relu(dl * sl_ref[...] + bl_ref[...][:, None, :])
    mg_ref[...] = dg * jax.nn.relu(dg * sg_ref[...] + bg_ref[...][:, None, :])


def _midmap(S_l, S_g, dinv2, b_l1, b_g1):
    return pl.pallas_call(
        _mid_body,
        grid=(NROW,),
        in_specs=[
            pl.BlockSpec((2, RT, 128), lambda i: (0, i, 0)),
            pl.BlockSpec((2, RT, 128), lambda i: (0, i, 0)),
            pl.BlockSpec((RT, 2), lambda i: (i, 0)),
            pl.BlockSpec((2, 128), lambda i: (0, 0)),
            pl.BlockSpec((2, 128), lambda i: (0, 0)),
        ],
        out_specs=[
            pl.BlockSpec((2, RT, 128), lambda i: (0, i, 0)),
            pl.BlockSpec((2, RT, 128), lambda i: (0, i, 0)),
        ],
        out_shape=[
            jax.ShapeDtypeStruct((2, NP, 128), jnp.float32),
            jax.ShapeDtypeStruct((2, NP, 128), jnp.float32),
        ],
    )(S_l, S_g, dinv2, b_l1, b_g1)


def _pool_body(s2l_ref, s2g_ref, dinv_ref, batch_ref, wl2_ref, wg2_ref, bl2_ref,
               bg2_ref, wf_ref, bf_ref, out_ref, pl_acc, pg_acc, cnt_acc):
    i = pl.program_id(0)
    batch_blk = batch_ref[0]
    gids = lax.broadcasted_iota(jnp.int32, (G, RTC), 0)
    mask = (gids == batch_blk).astype(jnp.float32)

    dl = dinv_ref[:, 0:1]
    dg = dinv_ref[:, 1:2]
    s2l = jnp.concatenate([s2l_ref[0], s2l_ref[1]], axis=1) * dl
    s2g = jnp.concatenate([s2g_ref[0], s2g_ref[1]], axis=1) * dg

    @pl.when(i == 0)
    def _():
        pl_acc[...] = jnp.zeros_like(pl_acc)
        pg_acc[...] = jnp.zeros_like(pg_acc)
        cnt_acc[...] = jnp.zeros_like(cnt_acc)

    pl_acc[...] += jnp.dot(mask, s2l, preferred_element_type=jnp.float32)
    pg_acc[...] += jnp.dot(mask, s2g, preferred_element_type=jnp.float32)
    cnt_acc[...] += jnp.sum(mask, axis=1, keepdims=True) * jnp.ones((G, 128), jnp.float32)

    @pl.when(i == NROWC - 1)
    def _():
        cnt = cnt_acc[:, 0:1]
        cmax = jnp.maximum(cnt, 1.0)
        nz = (cnt > 0.0).astype(jnp.float32)
        xl = jnp.dot(pl_acc[...] / cmax, wl2_ref[...],
                     preferred_element_type=jnp.float32) + bl2_ref[...] * nz
        xg = jnp.dot(pg_acc[...] / cmax, wg2_ref[...],
                     preferred_element_type=jnp.float32) + bg2_ref[...] * nz
        wf_top = wf_ref[0:128, :]
        wf_bot = wf_ref[128:256, :]
        fused = (jnp.dot(xl, wf_top, preferred_element_type=jnp.float32)
                 + jnp.dot(xg, wf_bot, preferred_element_type=jnp.float32)
                 + bf_ref[...])
        out_ref[...] = jax.nn.relu(fused)


def _pool_fuse(S2_l, S2_g, dinv2, batch3, W_l2, W_g2, b_l2, b_g2, W_fuse, b_fuse):
    return pl.pallas_call(
        _pool_body,
        grid=(NROWC,),
        in_specs=[
            pl.BlockSpec((2, RTC, 128), lambda i: (0, i, 0)),
            pl.BlockSpec((2, RTC, 128), lambda i: (0, i, 0)),
            pl.BlockSpec((RTC, 2), lambda i: (i, 0)),
            pl.BlockSpec((1, 1, RTC), lambda i: (i, 0, 0)),
            pl.BlockSpec((HID, 128), lambda i: (0, 0)),
            pl.BlockSpec((HID, 128), lambda i: (0, 0)),
            pl.BlockSpec((1, 128), lambda i: (0, 0)),
            pl.BlockSpec((1, 128), lambda i: (0, 0)),
            pl.BlockSpec((2 * OUT_D, 128), lambda i: (0, 0)),
            pl.BlockSpec((1, 128), lambda i: (0, 0)),
        ],
        out_specs=pl.BlockSpec((G, 128), lambda i: (0, 0)),
        out_shape=jax.ShapeDtypeStruct((G, OUT_D), jnp.float32),
        scratch_shapes=[
            pltpu.VMEM((G, HID), jnp.float32),
            pltpu.VMEM((G, HID), jnp.float32),
            pltpu.VMEM((G, 128), jnp.float32),
        ],
    )(S2_l, S2_g, dinv2, batch3, W_l2, W_g2, b_l2, b_g2, W_fuse, b_fuse)


def _prep_agg_idx(src, dst):
    srcp = jnp.concatenate([src, jnp.zeros((AGG_EPAD - E,), jnp.int32)])
    src2 = jnp.stack([srcp, srcp + NP]).reshape(NC * AGG_CHUNKS, CH)
    dstp = jnp.concatenate([dst, jnp.full((AGG_EPAD - E,), TRASH, jnp.int32)])
    return src2, dstp.reshape(AGG_CHUNKS, CH)


@jax.jit
def _run(x, edge_index_l, edge_index_g, batch, W_l1, b_l1, W_l2, b_l2,
         W_g1, b_g1, W_g2, b_g2, W_fuse, b_fuse):
    f32 = jnp.float32
    src2_l, dstp_l = _prep_agg_idx(edge_index_l[0], edge_index_l[1])
    src2_g, dstp_g = _prep_agg_idx(edge_index_g[0], edge_index_g[1])
    ones_tab = jnp.ones((2 * NP, 128), f32)
    deg_l = _agg_kernel(ones_tab, src2_l, dstp_l)
    deg_g = _agg_kernel(ones_tab, src2_g, dstp_g)

    x_p = jnp.pad(x, ((0, NP - N), (0, KPAD - IN_D)))
    w_cat = jnp.pad(jnp.concatenate([W_l1, W_g1], axis=1), ((0, KPAD - IN_D), (0, 0)))
    Hp, dinv2 = _matmul_scale(x_p, w_cat, deg_l, deg_g)

    S_l = _agg_kernel(Hp[0:2].reshape(2 * NP, 128), src2_l, dstp_l).reshape(2, NP, 128)
    S_g = _agg_kernel(Hp[2:4].reshape(2 * NP, 128), src2_g, dstp_g).reshape(2, NP, 128)

    Mp_l, Mp_g = _midmap(S_l, S_g, dinv2, b_l1.reshape(2, 128), b_g1.reshape(2, 128))

    S2_l = _agg_kernel(Mp_l.reshape(2 * NP, 128), src2_l, dstp_l).reshape(2, NP, 128)
    S2_g = _agg_kernel(Mp_g.reshape(2 * NP, 128), src2_g, dstp_g).reshape(2, NP, 128)

    batch3 = batch.reshape(NROWC, 1, RTC)
    out = _pool_fuse(S2_l, S2_g, dinv2, batch3, W_l2, W_g2,
                     b_l2.reshape(1, 128), b_g2.reshape(1, 128),
                     W_fuse, b_fuse.reshape(1, 128))
    return out


def kernel(x, edge_index_l, edge_index_g, batch, W_l1, b_l1, W_l2, b_l2,
           W_g1, b_g1, W_g2, b_g2, W_fuse, b_fuse):
    return _run(x, edge_index_l, edge_index_g, batch, W_l1, b_l1, W_l2, b_l2,
                W_g1, b_g1, W_g2, b_g2, W_fuse, b_fuse)


# speedup vs baseline: 5.7613x; 1.3674x over previous
import jax
import jax.numpy as jnp
from jax import lax
from jax.experimental import pallas as pl
from jax.experimental.pallas import tpu as pltpu
from jax.experimental.pallas import tpu_sc as plsc

N = 10000
E = 320000
G = 64
IN_D = 2063
HID = 256
OUT_D = 128

NC = 2
NS = 16
CH = 128

KPAD = 2176
NP = 10240
RT = 1024
NROW = NP // RT
RTC = 1000
NROWC = N // RTC
TRASH = N

AGG_TCH = 160
AGG_CHUNKS = AGG_TCH * NS
AGG_EPAD = AGG_CHUNKS * CH

RPT = NP // NS

_mesh = plsc.VectorSubcoreMesh(core_axis_name="c", subcore_axis_name="s",
                               num_cores=NC, num_subcores=NS)


NG = AGG_TCH // 8


def _agg_body(tab_hbm, src2_hbm, dst_hbm, out_hbm, src_v, dst_v, buf, acc,
              sem0, sem1):
    c = lax.axis_index("c")
    s = lax.axis_index("s")
    sems = (sem0, sem1)
    pltpu.sync_copy(tab_hbm.at[pl.ds(c * NP + s * RPT, RPT)], acc.at[pl.ds(s * RPT, RPT)])
    plsc.subcore_barrier()

    base = c * AGG_CHUNKS + s * AGG_TCH
    pltpu.sync_copy(src2_hbm.at[pl.ds(base, 8)], src_v.at[0])
    pltpu.sync_copy(dst_hbm.at[pl.ds(s * AGG_TCH, 8)], dst_v.at[0])
    pltpu.async_copy(tab_hbm.at[src_v.at[0].at[0]], buf.at[0], sem0)
    pltpu.async_copy(tab_hbm.at[src_v.at[0].at[1]], buf.at[1], sem1)

    @pl.loop(0, NG)
    def _grp(g):
        gp = g % 2
        gn = (g + 1) % 2

        @pl.when(g + 1 < NG)
        def _():
            pltpu.sync_copy(src2_hbm.at[pl.ds(base + (g + 1) * 8, 8)], src_v.at[gn])
            pltpu.sync_copy(dst_hbm.at[pl.ds(s * AGG_TCH + (g + 1) * 8, 8)], dst_v.at[gn])

        for k in range(8):
            p = k % 2
            pltpu.make_async_copy(tab_hbm.at[src_v.at[gp].at[k]], buf.at[p],
                                  sems[p]).wait()
            pltpu.sync_copy(buf.at[p], acc.at[dst_v.at[gp].at[k]], add=True)
            if k < 6:
                pltpu.async_copy(tab_hbm.at[src_v.at[gp].at[k + 2]], buf.at[p],
                                 sems[p])
            else:
                @pl.when(g + 1 < NG)
                def _():
                    pltpu.async_copy(tab_hbm.at[src_v.at[gn].at[k - 6]], buf.at[p],
                                     sems[p])

    plsc.subcore_barrier()
    pltpu.sync_copy(acc.at[pl.ds(s * RPT, RPT)], out_hbm.at[pl.ds(c * NP + s * RPT, RPT)])


_agg_kernel = pl.kernel(
    _agg_body, mesh=_mesh,
    out_type=jax.ShapeDtypeStruct((2 * NP, 128), jnp.float32),
    scratch_types=[
        pltpu.VMEM((2, 8, CH), jnp.int32),
        pltpu.VMEM((2, 8, CH), jnp.int32),
        pltpu.VMEM((2, CH, 128), jnp.float32),
        pltpu.VMEM_SHARED((NP, 128), jnp.float32),
        pltpu.SemaphoreType.DMA,
        pltpu.SemaphoreType.DMA,
    ],
)


DEG_HCH = 1280
DEG_TCH = 80


def _deg_body(dst_hbm, ones_hbm, mask_hbm, out_hbm, dst_v, ml_v, mg_v, acc):
    c = lax.axis_index("c")
    s = lax.axis_index("s")
    pltpu.sync_copy(mask_hbm.at[pl.ds(0, CH)], ml_v)
    pltpu.sync_copy(mask_hbm.at[pl.ds(CH, CH)], mg_v)
    pltpu.sync_copy(ones_hbm.at[pl.ds(c * NP + s * RPT, RPT)], acc.at[pl.ds(s * RPT, RPT)])
    plsc.subcore_barrier()
    for half, src_buf in ((0, ml_v), (1, mg_v)):

        @pl.loop(0, DEG_TCH // 8)
        def _grp(g):
            base = c * 2 * DEG_HCH + half * DEG_HCH + s * DEG_TCH + g * 8
            pltpu.sync_copy(dst_hbm.at[pl.ds(base, 8)], dst_v)
            for k in range(8):
                pltpu.sync_copy(src_buf, acc.at[dst_v.at[k]], add=True)

    plsc.subcore_barrier()
    pltpu.sync_copy(acc.at[pl.ds(s * RPT, RPT)], out_hbm.at[pl.ds(c * NP + s * RPT, RPT)])


_deg_kernel = pl.kernel(
    _deg_body, mesh=_mesh,
    out_type=jax.ShapeDtypeStruct((2 * NP, 128), jnp.float32),
    scratch_types=[
        pltpu.VMEM((8, CH), jnp.int32),
        pltpu.VMEM((CH, 128), jnp.float32),
        pltpu.VMEM((CH, 128), jnp.float32),
        pltpu.VMEM_SHARED((NP, 128), jnp.float32),
    ],
)


def _mm_body(x_ref, w_ref, h_ref):
    h_ref[...] = jnp.dot(x_ref[...], w_ref[...],
                         preferred_element_type=jnp.float32)[None]


def _matmul(x_p, w_cat):
    return pl.pallas_call(
        _mm_body,
        grid=(NROW, 4),
        in_specs=[
            pl.BlockSpec((RT, KPAD), lambda i, j: (i, 0)),
            pl.BlockSpec((KPAD, 128), lambda i, j: (0, j)),
        ],
        out_specs=pl.BlockSpec((1, RT, 128), lambda i, j: (j, i, 0)),
        out_shape=jax.ShapeDtypeStruct((4, NP, 128), jnp.float32),
    )(x_p, w_cat)


def _sc_body(h_ref, dega_ref, degb_ref, hp_ref, dinv_ref):
    deg_l = dega_ref[:, 0:1] + degb_ref[:, 0:1] - 1.0
    deg_g = dega_ref[:, 64:65] + degb_ref[:, 64:65] - 1.0
    dinv_l = lax.rsqrt(jnp.maximum(deg_l, 1.0))
    dinv_g = lax.rsqrt(jnp.maximum(deg_g, 1.0))
    dinv4 = jnp.stack([dinv_l, dinv_l, dinv_g, dinv_g])
    hp_ref[...] = h_ref[...] * dinv4
    dinv_ref[...] = jnp.concatenate([dinv_l, dinv_g], axis=1)


def _scale(H4, deg):
    return pl.pallas_call(
        _sc_body,
        grid=(NROW,),
        in_specs=[
            pl.BlockSpec((4, RT, 128), lambda i: (0, i, 0)),
            pl.BlockSpec((RT, 128), lambda i: (i, 0)),
            pl.BlockSpec((RT, 128), lambda i: (i + NROW, 0)),
        ],
        out_specs=[
            pl.BlockSpec((4, RT, 128), lambda i: (0, i, 0)),
            pl.BlockSpec((RT, 2), lambda i: (i, 0)),
        ],
        out_shape=[
            jax.ShapeDtypeStruct((4, NP, 128), jnp.float32),
            jax.ShapeDtypeStruct((NP, 2), jnp.float32),
        ],
    )(H4, deg, deg)


def _midmap(S, dinv2, b1, col):
    def body(s_ref, dinv_ref, b_ref, m_ref):
        d = dinv_ref[:, col:col + 1][None]
        m_ref[...] = d * jax.nn.relu(d * s_ref[...] + b_ref[...][:, None, :])

    return pl.pallas_call(
        body,
        grid=(NROW,),
        in_specs=[
            pl.BlockSpec((2, RT, 128), lambda i: (0, i, 0)),
            pl.BlockSpec((RT, 2), lambda i: (i, 0)),
            pl.BlockSpec((2, 128), lambda i: (0, 0)),
        ],
        out_specs=pl.BlockSpec((2, RT, 128), lambda i: (0, i, 0)),
        out_shape=jax.ShapeDtypeStruct((2, NP, 128), jnp.float32),
    )(S, dinv2, b1)


def _pool(S2, dinv2, batch3, col):
    def body(s2_ref, dinv_ref, batch_ref, p_ref, cnt_ref):
        i = pl.program_id(0)
        batch_blk = batch_ref[0]
        gids = lax.broadcasted_iota(jnp.int32, (G, RTC), 0)
        mask = (gids == batch_blk).astype(jnp.float32)
        d = dinv_ref[:, col:col + 1]
        scaled = jnp.concatenate([s2_ref[0], s2_ref[1]], axis=1) * d

        @pl.when(i == 0)
        def _():
            p_ref[...] = jnp.zeros_like(p_ref)
            cnt_ref[...] = jnp.zeros_like(cnt_ref)

        p_ref[...] += jnp.dot(mask, scaled, preferred_element_type=jnp.float32)
        cnt_ref[...] += (jnp.sum(mask, axis=1, keepdims=True)
                         * jnp.ones((G, 128), jnp.float32))

    return pl.pallas_call(
        body,
        grid=(NROWC,),
        in_specs=[
            pl.BlockSpec((2, RTC, 128), lambda i: (0, i, 0)),
            pl.BlockSpec((RTC, 2), lambda i: (i, 0)),
            pl.BlockSpec((1, 1, RTC), lambda i: (i, 0, 0)),
        ],
        out_specs=[
            pl.BlockSpec((G, HID), lambda i: (0, 0)),
            pl.BlockSpec((G, 128), lambda i: (0, 0)),
        ],
        out_shape=[
            jax.ShapeDtypeStruct((G, HID), jnp.float32),
            jax.ShapeDtypeStruct((G, 128), jnp.float32),
        ],
    )(S2, dinv2, batch3)


def _fuse_body(pl_ref, pg_ref, cnt_ref, wl2_ref, wg2_ref, bl2_ref, bg2_ref,
               wf_ref, bf_ref, out_ref):
    cnt = cnt_ref[:, 0:1]
    cmax = jnp.maximum(cnt, 1.0)
    nz = (cnt > 0.0).astype(jnp.float32)
    xl = jnp.dot(pl_ref[...] / cmax, wl2_ref[...],
                 preferred_element_type=jnp.float32) + bl2_ref[...] * nz
    xg = jnp.dot(pg_ref[...] / cmax, wg2_ref[...],
                 preferred_element_type=jnp.float32) + bg2_ref[...] * nz
    fused = (jnp.dot(xl, wf_ref[0:128, :], preferred_element_type=jnp.float32)
             + jnp.dot(xg, wf_ref[128:256, :], preferred_element_type=jnp.float32)
             + bf_ref[...])
    out_ref[...] = jax.nn.relu(fused)


def _fuse(P_l, P_g, cnt, W_l2, W_g2, b_l2, b_g2, W_fuse, b_fuse):
    return pl.pallas_call(
        _fuse_body,
        out_shape=jax.ShapeDtypeStruct((G, OUT_D), jnp.float32),
    )(P_l, P_g, cnt, W_l2, W_g2, b_l2, b_g2, W_fuse, b_fuse)


def _prep_agg_idx(src, dst):
    srcp = jnp.concatenate([src, jnp.zeros((AGG_EPAD - E,), jnp.int32)])
    src2 = jnp.stack([srcp, srcp + NP]).reshape(NC * AGG_CHUNKS, CH)
    dstp = jnp.concatenate([dst, jnp.full((AGG_EPAD - E,), TRASH, jnp.int32)])
    return src2, dstp.reshape(AGG_CHUNKS, CH)


@jax.jit
def _run(x, edge_index_l, edge_index_g, batch, W_l1, b_l1, W_l2, b_l2,
         W_g1, b_g1, W_g2, b_g2, W_fuse, b_fuse):
    f32 = jnp.float32
    x_p = jnp.pad(x, ((0, NP - N), (0, KPAD - IN_D)))
    w_cat = jnp.pad(jnp.concatenate([W_l1, W_g1], axis=1), ((0, KPAD - IN_D), (0, 0)))
    H4 = _matmul(x_p, w_cat)

    hE = E // NC
    def seg(dst, c):
        return jnp.concatenate([dst[c * hE:(c + 1) * hE],
                                jnp.full((DEG_HCH * CH - hE,), TRASH, jnp.int32)])
    dst_deg = jnp.concatenate(
        [seg(edge_index_l[1], 0), seg(edge_index_g[1], 0),
         seg(edge_index_l[1], 1), seg(edge_index_g[1], 1)]).reshape(NC * 2 * DEG_HCH, CH)
    ones_tab = jnp.ones((2 * NP, 128), f32)
    lane = jnp.arange(128)
    maskcat = jnp.concatenate(
        [jnp.broadcast_to((lane < 64).astype(f32), (CH, 128)),
         jnp.broadcast_to((lane >= 64).astype(f32), (CH, 128))])
    deg = _deg_kernel(dst_deg, ones_tab, maskcat)

    Hp, dinv2 = _scale(H4, deg)

    src2_l, dstp_l = _prep_agg_idx(edge_index_l[0], edge_index_l[1])
    src2_g, dstp_g = _prep_agg_idx(edge_index_g[0], edge_index_g[1])
    batch3 = batch.reshape(NROWC, 1, RTC)

    S_l = _agg_kernel(Hp[0:2].reshape(2 * NP, 128), src2_l, dstp_l).reshape(2, NP, 128)
    S_g = _agg_kernel(Hp[2:4].reshape(2 * NP, 128), src2_g, dstp_g).reshape(2, NP, 128)
    Mp_l = _midmap(S_l, dinv2, b_l1.reshape(2, 128), 0)
    S2_l = _agg_kernel(Mp_l.reshape(2 * NP, 128), src2_l, dstp_l).reshape(2, NP, 128)
    Mp_g = _midmap(S_g, dinv2, b_g1.reshape(2, 128), 1)
    S2_g = _agg_kernel(Mp_g.reshape(2 * NP, 128), src2_g, dstp_g).reshape(2, NP, 128)
    P_l, cnt = _pool(S2_l, dinv2, batch3, 0)
    P_g, _cnt2 = _pool(S2_g, dinv2, batch3, 1)

    return _fuse(P_l, P_g, cnt, W_l2, W_g2, b_l2.reshape(1, 128),
                 b_g2.reshape(1, 128), W_fuse, b_fuse.reshape(1, 128))


def kernel(x, edge_index_l, edge_index_g, batch, W_l1, b_l1, W_l2, b_l2,
           W_g1, b_g1, W_g2, b_g2, W_fuse, b_fuse):
    return _run(x, edge_index_l, edge_index_g, batch, W_l1, b_l1, W_l2, b_l2,
                W_g1, b_g1, W_g2, b_g2, W_fuse, b_fuse)


# speedup vs baseline: 6.6076x; 1.1469x over previous
import jax
import jax.numpy as jnp
from jax import lax
from jax.experimental import pallas as pl
from jax.experimental.pallas import tpu as pltpu
from jax.experimental.pallas import tpu_sc as plsc

N = 10000
E = 320000
G = 64
IN_D = 2063
HID = 256
OUT_D = 128

NC = 2
NS = 16
CH = 128

KPAD = 2176
NP = 10240
RT = 1024
NROW = NP // RT
RTC = 1000
NROWC = N // RTC
TRASH = N

AGG_TCH = 160
AGG_CHUNKS = AGG_TCH * NS
AGG_EPAD = AGG_CHUNKS * CH

RPT = NP // NS

_mesh = plsc.VectorSubcoreMesh(core_axis_name="c", subcore_axis_name="s",
                               num_cores=NC, num_subcores=NS)


NG = AGG_TCH // 8


def _agg_body(tab_hbm, src2_hbm, dst_hbm, out_hbm, src_v, dst_v, buf, acc,
              sem0, sem1):
    c = lax.axis_index("c")
    s = lax.axis_index("s")
    sems = (sem0, sem1)
    pltpu.sync_copy(tab_hbm.at[pl.ds(c * NP + s * RPT, RPT)], acc.at[pl.ds(s * RPT, RPT)])
    plsc.subcore_barrier()

    base = c * AGG_CHUNKS + s * AGG_TCH
    pltpu.sync_copy(src2_hbm.at[pl.ds(base, 8)], src_v.at[0])
    pltpu.sync_copy(dst_hbm.at[pl.ds(s * AGG_TCH, 8)], dst_v.at[0])
    pltpu.async_copy(tab_hbm.at[src_v.at[0].at[0]], buf.at[0], sem0)
    pltpu.async_copy(tab_hbm.at[src_v.at[0].at[1]], buf.at[1], sem1)

    @pl.loop(0, NG)
    def _grp(g):
        gp = g % 2
        gn = (g + 1) % 2

        @pl.when(g + 1 < NG)
        def _():
            pltpu.sync_copy(src2_hbm.at[pl.ds(base + (g + 1) * 8, 8)], src_v.at[gn])
            pltpu.sync_copy(dst_hbm.at[pl.ds(s * AGG_TCH + (g + 1) * 8, 8)], dst_v.at[gn])

        for k in range(8):
            p = k % 2
            pltpu.make_async_copy(tab_hbm.at[src_v.at[gp].at[k]], buf.at[p],
                                  sems[p]).wait()
            pltpu.sync_copy(buf.at[p], acc.at[dst_v.at[gp].at[k]], add=True)
            if k < 6:
                pltpu.async_copy(tab_hbm.at[src_v.at[gp].at[k + 2]], buf.at[p],
                                 sems[p])
            else:
                @pl.when(g + 1 < NG)
                def _():
                    pltpu.async_copy(tab_hbm.at[src_v.at[gn].at[k - 6]], buf.at[p],
                                     sems[p])

    plsc.subcore_barrier()
    pltpu.sync_copy(acc.at[pl.ds(s * RPT, RPT)], out_hbm.at[pl.ds(c * NP + s * RPT, RPT)])


_agg_kernel = pl.kernel(
    _agg_body, mesh=_mesh,
    out_type=jax.ShapeDtypeStruct((2 * NP, 128), jnp.float32),
    scratch_types=[
        pltpu.VMEM((2, 8, CH), jnp.int32),
        pltpu.VMEM((2, 8, CH), jnp.int32),
        pltpu.VMEM((2, CH, 128), jnp.float32),
        pltpu.VMEM_SHARED((NP, 128), jnp.float32),
        pltpu.SemaphoreType.DMA,
        pltpu.SemaphoreType.DMA,
    ],
)


DEG_HCH = 1280
DEG_TCH = 80


def _deg_body(dst_hbm, mask_hbm, out_hbm, dst_v, ml_v, mg_v, acc):
    c = lax.axis_index("c")
    s = lax.axis_index("s")
    pltpu.sync_copy(mask_hbm.at[pl.ds(0, CH)], ml_v)
    pltpu.sync_copy(mask_hbm.at[pl.ds(CH, CH)], mg_v)

    @pl.loop(0, RPT // CH)
    def _init(q):
        pltpu.sync_copy(ml_v, acc.at[pl.ds(s * RPT + q * CH, CH)])

    plsc.subcore_barrier()
    for half, src_buf in ((0, ml_v), (1, mg_v)):

        @pl.loop(0, DEG_TCH // 8)
        def _grp(g):
            base = c * 2 * DEG_HCH + half * DEG_HCH + s * DEG_TCH + g * 8
            pltpu.sync_copy(dst_hbm.at[pl.ds(base, 8)], dst_v)
            for k in range(8):
                pltpu.sync_copy(src_buf, acc.at[dst_v.at[k]], add=True)

    plsc.subcore_barrier()
    pltpu.sync_copy(acc.at[pl.ds(s * RPT, RPT)], out_hbm.at[pl.ds(c * NP + s * RPT, RPT)])


_deg_kernel = pl.kernel(
    _deg_body, mesh=_mesh,
    out_type=jax.ShapeDtypeStruct((2 * NP, 128), jnp.float32),
    scratch_types=[
        pltpu.VMEM((8, CH), jnp.int32),
        pltpu.VMEM((CH, 128), jnp.float32),
        pltpu.VMEM((CH, 128), jnp.float32),
        pltpu.VMEM_SHARED((NP, 128), jnp.float32),
    ],
)


def _mm_body(x_ref, w_ref, h_ref):
    h_ref[...] = jnp.dot(x_ref[...], w_ref[...],
                         preferred_element_type=jnp.float32)[None]


def _matmul(x_p, w_cat):
    return pl.pallas_call(
        _mm_body,
        grid=(NROW, 4),
        in_specs=[
            pl.BlockSpec((RT, KPAD), lambda i, j: (i, 0)),
            pl.BlockSpec((KPAD, 128), lambda i, j: (0, j)),
        ],
        out_specs=pl.BlockSpec((1, RT, 128), lambda i, j: (j, i, 0)),
        out_shape=jax.ShapeDtypeStruct((4, NP, 128), jnp.float32),
    )(x_p, w_cat)


def _sc_body(h_ref, dega_ref, degb_ref, hpl_ref, hpg_ref, dinv_ref):
    deg_l = dega_ref[:, 0:1] + degb_ref[:, 0:1] - 1.0
    deg_g = dega_ref[:, 64:65] + degb_ref[:, 64:65] + 1.0
    dinv_l = lax.rsqrt(jnp.maximum(deg_l, 1.0))
    dinv_g = lax.rsqrt(jnp.maximum(deg_g, 1.0))
    hpl_ref[...] = h_ref[0:2] * dinv_l[None]
    hpg_ref[...] = h_ref[2:4] * dinv_g[None]
    dinv_ref[...] = jnp.concatenate([dinv_l, dinv_g], axis=1)


def _scale(H4, deg):
    return pl.pallas_call(
        _sc_body,
        grid=(NROW,),
        in_specs=[
            pl.BlockSpec((4, RT, 128), lambda i: (0, i, 0)),
            pl.BlockSpec((RT, 128), lambda i: (i, 0)),
            pl.BlockSpec((RT, 128), lambda i: (i + NROW, 0)),
        ],
        out_specs=[
            pl.BlockSpec((2, RT, 128), lambda i: (0, i, 0)),
            pl.BlockSpec((2, RT, 128), lambda i: (0, i, 0)),
            pl.BlockSpec((RT, 2), lambda i: (i, 0)),
        ],
        out_shape=[
            jax.ShapeDtypeStruct((2, NP, 128), jnp.float32),
            jax.ShapeDtypeStruct((2, NP, 128), jnp.float32),
            jax.ShapeDtypeStruct((NP, 2), jnp.float32),
        ],
    )(H4, deg, deg)


def _midmap(S, dinv2, b1, col):
    def body(s_ref, dinv_ref, b_ref, m_ref):
        d = dinv_ref[:, col:col + 1][None]
        m_ref[...] = d * jax.nn.relu(d * s_ref[...] + b_ref[...][:, None, :])

    return pl.pallas_call(
        body,
        grid=(NROW,),
        in_specs=[
            pl.BlockSpec((2, RT, 128), lambda i: (0, i, 0)),
            pl.BlockSpec((RT, 2), lambda i: (i, 0)),
            pl.BlockSpec((2, 128), lambda i: (0, 0)),
        ],
        out_specs=pl.BlockSpec((2, RT, 128), lambda i: (0, i, 0)),
        out_shape=jax.ShapeDtypeStruct((2, NP, 128), jnp.float32),
    )(S, dinv2, b1)


def _pool(S2, dinv2, batch3, col):
    def body(s2_ref, dinv_ref, batch_ref, p_ref, cnt_ref):
        i = pl.program_id(0)
        batch_blk = batch_ref[0]
        gids = lax.broadcasted_iota(jnp.int32, (G, RTC), 0)
        mask = (gids == batch_blk).astype(jnp.float32)
        d = dinv_ref[:, col:col + 1]
        scaled = jnp.concatenate([s2_ref[0], s2_ref[1]], axis=1) * d

        @pl.when(i == 0)
        def _():
            p_ref[...] = jnp.zeros_like(p_ref)
            cnt_ref[...] = jnp.zeros_like(cnt_ref)

        p_ref[...] += jnp.dot(mask, scaled, preferred_element_type=jnp.float32)
        cnt_ref[...] += (jnp.sum(mask, axis=1, keepdims=True)
                         * jnp.ones((G, 128), jnp.float32))

    return pl.pallas_call(
        body,
        grid=(NROWC,),
        in_specs=[
            pl.BlockSpec((2, RTC, 128), lambda i: (0, i, 0)),
            pl.BlockSpec((RTC, 2), lambda i: (i, 0)),
            pl.BlockSpec((1, 1, RTC), lambda i: (i, 0, 0)),
        ],
        out_specs=[
            pl.BlockSpec((G, HID), lambda i: (0, 0)),
            pl.BlockSpec((G, 128), lambda i: (0, 0)),
        ],
        out_shape=[
            jax.ShapeDtypeStruct((G, HID), jnp.float32),
            jax.ShapeDtypeStruct((G, 128), jnp.float32),
        ],
    )(S2, dinv2, batch3)


def _fuse_body(pl_ref, pg_ref, cnt_ref, wl2_ref, wg2_ref, bl2_ref, bg2_ref,
               wf_ref, bf_ref, out_ref):
    cnt = cnt_ref[:, 0:1]
    cmax = jnp.maximum(cnt, 1.0)
    nz = (cnt > 0.0).astype(jnp.float32)
    xl = jnp.dot(pl_ref[...] / cmax, wl2_ref[...],
                 preferred_element_type=jnp.float32) + bl2_ref[...] * nz
    xg = jnp.dot(pg_ref[...] / cmax, wg2_ref[...],
                 preferred_element_type=jnp.float32) + bg2_ref[...] * nz
    fused = (jnp.dot(xl, wf_ref[0:128, :], preferred_element_type=jnp.float32)
             + jnp.dot(xg, wf_ref[128:256, :], preferred_element_type=jnp.float32)
             + bf_ref[...])
    out_ref[...] = jax.nn.relu(fused)


def _fuse(P_l, P_g, cnt, W_l2, W_g2, b_l2, b_g2, W_fuse, b_fuse):
    return pl.pallas_call(
        _fuse_body,
        out_shape=jax.ShapeDtypeStruct((G, OUT_D), jnp.float32),
    )(P_l, P_g, cnt, W_l2, W_g2, b_l2, b_g2, W_fuse, b_fuse)


def _padx_body(x_ref, o_ref):
    o_ref[...] = jnp.concatenate(
        [x_ref[...], jnp.zeros((RT, KPAD - IN_D), jnp.float32)], axis=1)


def _padx(x):
    return pl.pallas_call(
        _padx_body,
        grid=(NROW,),
        in_specs=[pl.BlockSpec((RT, IN_D), lambda i: (i, 0))],
        out_specs=pl.BlockSpec((RT, KPAD), lambda i: (i, 0)),
        out_shape=jax.ShapeDtypeStruct((NP, KPAD), jnp.float32),
    )(x)


def _prepw_body(wl_ref, wg_ref, o_ref):
    w = jnp.concatenate([wl_ref[...], wg_ref[...]], axis=1)
    o_ref[...] = jnp.concatenate(
        [w, jnp.zeros((KPAD - IN_D, 2 * HID), jnp.float32)], axis=0)


def _prepw(W_l1, W_g1):
    return pl.pallas_call(
        _prepw_body,
        out_shape=jax.ShapeDtypeStruct((KPAD, 2 * HID), jnp.float32),
    )(W_l1, W_g1)


ECH = E // CH
HCH = ECH // NC


def _prepidx_body(il_ref, ig_ref, s2l_ref, s2g_ref, dl_ref, dg_ref, dd_ref):
    i32 = jnp.int32

    def srcfill(srow):
        pad = jnp.zeros((AGG_CHUNKS - ECH, CH), i32)
        return jnp.concatenate(
            [srow, pad, srow + NP, pad + NP], axis=0)

    def dstfill(drow):
        pad = jnp.full((AGG_CHUNKS - ECH, CH), TRASH, i32)
        return jnp.concatenate([drow, pad], axis=0)

    il_s, il_d = il_ref[0], il_ref[1]
    ig_s, ig_d = ig_ref[0], ig_ref[1]
    s2l_ref[...] = srcfill(il_s)
    s2g_ref[...] = srcfill(ig_s)
    dl_ref[...] = dstfill(il_d)
    dg_ref[...] = dstfill(ig_d)
    dpad = jnp.full((DEG_HCH - HCH, CH), TRASH, i32)
    dd_ref[...] = jnp.concatenate(
        [il_d[0:HCH], dpad, ig_d[0:HCH], dpad,
         il_d[HCH:ECH], dpad, ig_d[HCH:ECH], dpad], axis=0)


def _prepidx(eil, eig):
    return pl.pallas_call(
        _prepidx_body,
        out_shape=[
            jax.ShapeDtypeStruct((NC * AGG_CHUNKS, CH), jnp.int32),
            jax.ShapeDtypeStruct((NC * AGG_CHUNKS, CH), jnp.int32),
            jax.ShapeDtypeStruct((AGG_CHUNKS, CH), jnp.int32),
            jax.ShapeDtypeStruct((AGG_CHUNKS, CH), jnp.int32),
            jax.ShapeDtypeStruct((NC * 2 * DEG_HCH, CH), jnp.int32),
        ],
    )(eil, eig)


@jax.jit
def _run(x, edge_index_l, edge_index_g, batch, W_l1, b_l1, W_l2, b_l2,
         W_g1, b_g1, W_g2, b_g2, W_fuse, b_fuse):
    f32 = jnp.float32
    x_p = _padx(x)
    w_cat = _prepw(W_l1, W_g1)
    src2_l, src2_g, dstp_l, dstp_g, dst_deg = _prepidx(
        edge_index_l.reshape(2, ECH, CH), edge_index_g.reshape(2, ECH, CH))

    H4 = _matmul(x_p, w_cat)

    lane = jnp.arange(128)
    maskcat = jnp.concatenate(
        [jnp.broadcast_to((lane < 64).astype(f32), (CH, 128)),
         jnp.broadcast_to((lane >= 64).astype(f32), (CH, 128))])
    deg = _deg_kernel(dst_deg, maskcat)

    Hp_l, Hp_g, dinv2 = _scale(H4, deg)
    batch3 = batch.reshape(NROWC, 1, RTC)

    S_l = _agg_kernel(Hp_l.reshape(2 * NP, 128), src2_l, dstp_l).reshape(2, NP, 128)
    S_g = _agg_kernel(Hp_g.reshape(2 * NP, 128), src2_g, dstp_g).reshape(2, NP, 128)
    Mp_l = _midmap(S_l, dinv2, b_l1.reshape(2, 128), 0)
    S2_l = _agg_kernel(Mp_l.reshape(2 * NP, 128), src2_l, dstp_l).reshape(2, NP, 128)
    Mp_g = _midmap(S_g, dinv2, b_g1.reshape(2, 128), 1)
    S2_g = _agg_kernel(Mp_g.reshape(2 * NP, 128), src2_g, dstp_g).reshape(2, NP, 128)
    P_l, cnt = _pool(S2_l, dinv2, batch3, 0)
    P_g, _cnt2 = _pool(S2_g, dinv2, batch3, 1)

    return _fuse(P_l, P_g, cnt, W_l2, W_g2, b_l2.reshape(1, 128),
                 b_g2.reshape(1, 128), W_fuse, b_fuse.reshape(1, 128))


def kernel(x, edge_index_l, edge_index_g, batch, W_l1, b_l1, W_l2, b_l2,
           W_g1, b_g1, W_g2, b_g2, W_fuse, b_fuse):
    return _run(x, edge_index_l, edge_index_g, batch, W_l1, b_l1, W_l2, b_l2,
                W_g1, b_g1, W_g2, b_g2, W_fuse, b_fuse)


# speedup vs baseline: 6.6831x; 1.0114x over previous
import jax
import jax.numpy as jnp
from jax import lax
from jax.experimental import pallas as pl
from jax.experimental.pallas import tpu as pltpu
from jax.experimental.pallas import tpu_sc as plsc

N = 10000
E = 320000
G = 64
IN_D = 2063
HID = 256
OUT_D = 128

NC = 2
NS = 16
CH = 128

KPAD = 2176
NP = 10240
RT = 1024
NROW = NP // RT
RTC = 1000
NROWC = N // RTC
TRASH = N

AGG_TCH = 160
AGG_CHUNKS = AGG_TCH * NS
AGG_EPAD = AGG_CHUNKS * CH

RPT = NP // NS

_mesh = plsc.VectorSubcoreMesh(core_axis_name="c", subcore_axis_name="s",
                               num_cores=NC, num_subcores=NS)


NG = AGG_TCH // 8


def _agg_body(tab_hbm, src2_hbm, dst_hbm, out_hbm, src_v, dst_v, buf, acc,
              sem0, sem1):
    c = lax.axis_index("c")
    s = lax.axis_index("s")
    sems = (sem0, sem1)
    pltpu.sync_copy(tab_hbm.at[pl.ds(c * NP + s * RPT, RPT)], acc.at[pl.ds(s * RPT, RPT)])
    plsc.subcore_barrier()

    base = c * AGG_CHUNKS + s * AGG_TCH
    pltpu.sync_copy(src2_hbm.at[pl.ds(base, 8)], src_v.at[0])
    pltpu.sync_copy(dst_hbm.at[pl.ds(s * AGG_TCH, 8)], dst_v.at[0])
    pltpu.async_copy(tab_hbm.at[src_v.at[0].at[0]], buf.at[0], sem0)
    pltpu.async_copy(tab_hbm.at[src_v.at[0].at[1]], buf.at[1], sem1)

    @pl.loop(0, NG)
    def _grp(g):
        gp = g % 2
        gn = (g + 1) % 2

        @pl.when(g + 1 < NG)
        def _():
            pltpu.sync_copy(src2_hbm.at[pl.ds(base + (g + 1) * 8, 8)], src_v.at[gn])
            pltpu.sync_copy(dst_hbm.at[pl.ds(s * AGG_TCH + (g + 1) * 8, 8)], dst_v.at[gn])

        for k in range(8):
            p = k % 2
            pltpu.make_async_copy(tab_hbm.at[src_v.at[gp].at[k]], buf.at[p],
                                  sems[p]).wait()
            pltpu.sync_copy(buf.at[p], acc.at[dst_v.at[gp].at[k]], add=True)
            if k < 6:
                pltpu.async_copy(tab_hbm.at[src_v.at[gp].at[k + 2]], buf.at[p],
                                 sems[p])
            else:
                @pl.when(g + 1 < NG)
                def _():
                    pltpu.async_copy(tab_hbm.at[src_v.at[gn].at[k - 6]], buf.at[p],
                                     sems[p])

    plsc.subcore_barrier()
    pltpu.sync_copy(acc.at[pl.ds(s * RPT, RPT)], out_hbm.at[pl.ds(c * NP + s * RPT, RPT)])


_agg_kernel = pl.kernel(
    _agg_body, mesh=_mesh,
    out_type=jax.ShapeDtypeStruct((2 * NP, 128), jnp.float32),
    scratch_types=[
        pltpu.VMEM((2, 8, CH), jnp.int32),
        pltpu.VMEM((2, 8, CH), jnp.int32),
        pltpu.VMEM((2, CH, 128), jnp.float32),
        pltpu.VMEM_SHARED((NP, 128), jnp.float32),
        pltpu.SemaphoreType.DMA,
        pltpu.SemaphoreType.DMA,
    ],
)


DEG_HCH = 1280
DEG_TCH = 80


def _deg_body(dst_hbm, mask_hbm, out_hbm, dst_v, ml_v, mg_v, acc):
    c = lax.axis_index("c")
    s = lax.axis_index("s")
    pltpu.sync_copy(mask_hbm.at[pl.ds(0, CH)], ml_v)
    pltpu.sync_copy(mask_hbm.at[pl.ds(CH, CH)], mg_v)

    @pl.loop(0, RPT // CH)
    def _init(q):
        pltpu.sync_copy(ml_v, acc.at[pl.ds(s * RPT + q * CH, CH)])

    plsc.subcore_barrier()
    for half, src_buf in ((0, ml_v), (1, mg_v)):

        @pl.loop(0, DEG_TCH // 8)
        def _grp(g):
            base = c * 2 * DEG_HCH + half * DEG_HCH + s * DEG_TCH + g * 8
            pltpu.sync_copy(dst_hbm.at[pl.ds(base, 8)], dst_v)
            for k in range(8):
                pltpu.sync_copy(src_buf, acc.at[dst_v.at[k]], add=True)

    plsc.subcore_barrier()
    pltpu.sync_copy(acc.at[pl.ds(s * RPT, RPT)], out_hbm.at[pl.ds(c * NP + s * RPT, RPT)])


_deg_kernel = pl.kernel(
    _deg_body, mesh=_mesh,
    out_type=jax.ShapeDtypeStruct((2 * NP, 128), jnp.float32),
    scratch_types=[
        pltpu.VMEM((8, CH), jnp.int32),
        pltpu.VMEM((CH, 128), jnp.float32),
        pltpu.VMEM((CH, 128), jnp.float32),
        pltpu.VMEM_SHARED((NP, 128), jnp.float32),
    ],
)


def _mm_body(x_ref, w_ref, h_ref):
    h_ref[...] = jnp.dot(x_ref[...], w_ref[...],
                         preferred_element_type=jnp.float32)[None]


def _matmul(x_p, w_cat):
    return pl.pallas_call(
        _mm_body,
        grid=(NROW, 4),
        in_specs=[
            pl.BlockSpec((RT, KPAD), lambda i, j: (i, 0)),
            pl.BlockSpec((KPAD, 128), lambda i, j: (0, j)),
        ],
        out_specs=pl.BlockSpec((1, RT, 128), lambda i, j: (j, i, 0)),
        out_shape=jax.ShapeDtypeStruct((4, NP, 128), jnp.float32),
    )(x_p, w_cat)


def _sc_body(h_ref, dega_ref, degb_ref, hpl_ref, hpg_ref, dinv_ref):
    deg_l = dega_ref[:, 0:1] + degb_ref[:, 0:1] - 1.0
    deg_g = dega_ref[:, 64:65] + degb_ref[:, 64:65] + 1.0
    dinv_l = lax.rsqrt(jnp.maximum(deg_l, 1.0))
    dinv_g = lax.rsqrt(jnp.maximum(deg_g, 1.0))
    hpl_ref[...] = h_ref[0:2] * dinv_l[None]
    hpg_ref[...] = h_ref[2:4] * dinv_g[None]
    dinv_ref[...] = jnp.concatenate([dinv_l, dinv_g], axis=1)


def _scale(H4, deg):
    return pl.pallas_call(
        _sc_body,
        grid=(NROW,),
        in_specs=[
            pl.BlockSpec((4, RT, 128), lambda i: (0, i, 0)),
            pl.BlockSpec((RT, 128), lambda i: (i, 0)),
            pl.BlockSpec((RT, 128), lambda i: (i + NROW, 0)),
        ],
        out_specs=[
            pl.BlockSpec((2, RT, 128), lambda i: (0, i, 0)),
            pl.BlockSpec((2, RT, 128), lambda i: (0, i, 0)),
            pl.BlockSpec((RT, 2), lambda i: (i, 0)),
        ],
        out_shape=[
            jax.ShapeDtypeStruct((2, NP, 128), jnp.float32),
            jax.ShapeDtypeStruct((2, NP, 128), jnp.float32),
            jax.ShapeDtypeStruct((NP, 2), jnp.float32),
        ],
    )(H4, deg, deg)


def _midmap(S, dinv2, b1, col):
    def body(s_ref, dinv_ref, b_ref, m_ref):
        d = dinv_ref[:, col:col + 1][None]
        m_ref[...] = d * jax.nn.relu(d * s_ref[...] + b_ref[...][:, None, :])

    return pl.pallas_call(
        body,
        grid=(NROW,),
        in_specs=[
            pl.BlockSpec((2, RT, 128), lambda i: (0, i, 0)),
            pl.BlockSpec((RT, 2), lambda i: (i, 0)),
            pl.BlockSpec((2, 128), lambda i: (0, 0)),
        ],
        out_specs=pl.BlockSpec((2, RT, 128), lambda i: (0, i, 0)),
        out_shape=jax.ShapeDtypeStruct((2, NP, 128), jnp.float32),
    )(S, dinv2, b1)


def _pool(S2, dinv2, batch3, col):
    def body(s2_ref, dinv_ref, batch_ref, p_ref, cnt_ref):
        i = pl.program_id(0)
        batch_blk = batch_ref[0]
        gids = lax.broadcasted_iota(jnp.int32, (G, RTC), 0)
        mask = (gids == batch_blk).astype(jnp.float32)
        d = dinv_ref[:, col:col + 1]
        scaled = jnp.concatenate([s2_ref[0], s2_ref[1]], axis=1) * d

        @pl.when(i == 0)
        def _():
            p_ref[...] = jnp.zeros_like(p_ref)
            cnt_ref[...] = jnp.zeros_like(cnt_ref)

        p_ref[...] += jnp.dot(mask, scaled, preferred_element_type=jnp.float32)
        cnt_ref[...] += (jnp.sum(mask, axis=1, keepdims=True)
                         * jnp.ones((G, 128), jnp.float32))

    return pl.pallas_call(
        body,
        grid=(NROWC,),
        in_specs=[
            pl.BlockSpec((2, RTC, 128), lambda i: (0, i, 0)),
            pl.BlockSpec((RTC, 2), lambda i: (i, 0)),
            pl.BlockSpec((1, 1, RTC), lambda i: (i, 0, 0)),
        ],
        out_specs=[
            pl.BlockSpec((G, HID), lambda i: (0, 0)),
            pl.BlockSpec((G, 128), lambda i: (0, 0)),
        ],
        out_shape=[
            jax.ShapeDtypeStruct((G, HID), jnp.float32),
            jax.ShapeDtypeStruct((G, 128), jnp.float32),
        ],
    )(S2, dinv2, batch3)


def _fuse_body(pl_ref, pg_ref, cnt_ref, wl2_ref, wg2_ref, bl2_ref, bg2_ref,
               wf_ref, bf_ref, out_ref):
    cnt = cnt_ref[:, 0:1]
    cmax = jnp.maximum(cnt, 1.0)
    nz = (cnt > 0.0).astype(jnp.float32)
    xl = jnp.dot(pl_ref[...] / cmax, wl2_ref[...],
                 preferred_element_type=jnp.float32) + bl2_ref[...] * nz
    xg = jnp.dot(pg_ref[...] / cmax, wg2_ref[...],
                 preferred_element_type=jnp.float32) + bg2_ref[...] * nz
    fused = (jnp.dot(xl, wf_ref[0:128, :], preferred_element_type=jnp.float32)
             + jnp.dot(xg, wf_ref[128:256, :], preferred_element_type=jnp.float32)
             + bf_ref[...])
    out_ref[...] = jax.nn.relu(fused)


def _fuse(P_l, P_g, cnt, W_l2, W_g2, b_l2, b_g2, W_fuse, b_fuse):
    return pl.pallas_call(
        _fuse_body,
        out_shape=jax.ShapeDtypeStruct((G, OUT_D), jnp.float32),
    )(P_l, P_g, cnt, W_l2, W_g2, b_l2, b_g2, W_fuse, b_fuse)


def _padx_body(x_ref, o_ref):
    o_ref[...] = jnp.concatenate(
        [x_ref[...].astype(jnp.bfloat16),
         jnp.zeros((RT, KPAD - IN_D), jnp.bfloat16)], axis=1)


def _padx(x):
    return pl.pallas_call(
        _padx_body,
        grid=(NROW,),
        in_specs=[pl.BlockSpec((RT, IN_D), lambda i: (i, 0))],
        out_specs=pl.BlockSpec((RT, KPAD), lambda i: (i, 0)),
        out_shape=jax.ShapeDtypeStruct((NP, KPAD), jnp.bfloat16),
    )(x)


def _prepw_body(wl_ref, wg_ref, o_ref):
    w = jnp.concatenate([wl_ref[...], wg_ref[...]], axis=1).astype(jnp.bfloat16)
    o_ref[...] = jnp.concatenate(
        [w, jnp.zeros((KPAD - IN_D, 2 * HID), jnp.bfloat16)], axis=0)


def _prepw(W_l1, W_g1):
    return pl.pallas_call(
        _prepw_body,
        out_shape=jax.ShapeDtypeStruct((KPAD, 2 * HID), jnp.bfloat16),
    )(W_l1, W_g1)


ECH = E // CH
HCH = ECH // NC


def _prepidx_body(il_ref, ig_ref, s2l_ref, s2g_ref, dl_ref, dg_ref, dd_ref):
    i32 = jnp.int32

    def srcfill(srow):
        pad = jnp.zeros((AGG_CHUNKS - ECH, CH), i32)
        return jnp.concatenate(
            [srow, pad, srow + NP, pad + NP], axis=0)

    def dstfill(drow):
        pad = jnp.full((AGG_CHUNKS - ECH, CH), TRASH, i32)
        return jnp.concatenate([drow, pad], axis=0)

    il_s, il_d = il_ref[0], il_ref[1]
    ig_s, ig_d = ig_ref[0], ig_ref[1]
    s2l_ref[...] = srcfill(il_s)
    s2g_ref[...] = srcfill(ig_s)
    dl_ref[...] = dstfill(il_d)
    dg_ref[...] = dstfill(ig_d)
    dpad = jnp.full((DEG_HCH - HCH, CH), TRASH, i32)
    dd_ref[...] = jnp.concatenate(
        [il_d[0:HCH], dpad, ig_d[0:HCH], dpad,
         il_d[HCH:ECH], dpad, ig_d[HCH:ECH], dpad], axis=0)


def _prepidx(eil, eig):
    return pl.pallas_call(
        _prepidx_body,
        out_shape=[
            jax.ShapeDtypeStruct((NC * AGG_CHUNKS, CH), jnp.int32),
            jax.ShapeDtypeStruct((NC * AGG_CHUNKS, CH), jnp.int32),
            jax.ShapeDtypeStruct((AGG_CHUNKS, CH), jnp.int32),
            jax.ShapeDtypeStruct((AGG_CHUNKS, CH), jnp.int32),
            jax.ShapeDtypeStruct((NC * 2 * DEG_HCH, CH), jnp.int32),
        ],
    )(eil, eig)


@jax.jit
def _run(x, edge_index_l, edge_index_g, batch, W_l1, b_l1, W_l2, b_l2,
         W_g1, b_g1, W_g2, b_g2, W_fuse, b_fuse):
    f32 = jnp.float32
    x_p = _padx(x)
    w_cat = _prepw(W_l1, W_g1)
    src2_l, src2_g, dstp_l, dstp_g, dst_deg = _prepidx(
        edge_index_l.reshape(2, ECH, CH), edge_index_g.reshape(2, ECH, CH))

    H4 = _matmul(x_p, w_cat)

    lane = jnp.arange(128)
    maskcat = jnp.concatenate(
        [jnp.broadcast_to((lane < 64).astype(f32), (CH, 128)),
         jnp.broadcast_to((lane >= 64).astype(f32), (CH, 128))])
    deg = _deg_kernel(dst_deg, maskcat)

    Hp_l, Hp_g, dinv2 = _scale(H4, deg)
    batch3 = batch.reshape(NROWC, 1, RTC)

    S_l = _agg_kernel(Hp_l.reshape(2 * NP, 128), src2_l, dstp_l).reshape(2, NP, 128)
    S_g = _agg_kernel(Hp_g.reshape(2 * NP, 128), src2_g, dstp_g).reshape(2, NP, 128)
    Mp_l = _midmap(S_l, dinv2, b_l1.reshape(2, 128), 0)
    S2_l = _agg_kernel(Mp_l.reshape(2 * NP, 128), src2_l, dstp_l).reshape(2, NP, 128)
    Mp_g = _midmap(S_g, dinv2, b_g1.reshape(2, 128), 1)
    S2_g = _agg_kernel(Mp_g.reshape(2 * NP, 128), src2_g, dstp_g).reshape(2, NP, 128)
    P_l, cnt = _pool(S2_l, dinv2, batch3, 0)
    P_g, _cnt2 = _pool(S2_g, dinv2, batch3, 1)

    return _fuse(P_l, P_g, cnt, W_l2, W_g2, b_l2.reshape(1, 128),
                 b_g2.reshape(1, 128), W_fuse, b_fuse.reshape(1, 128))


def kernel(x, edge_index_l, edge_index_g, batch, W_l1, b_l1, W_l2, b_l2,
           W_g1, b_g1, W_g2, b_g2, W_fuse, b_fuse):
    return _run(x, edge_index_l, edge_index_g, batch, W_l1, b_l1, W_l2, b_l2,
                W_g1, b_g1, W_g2, b_g2, W_fuse, b_fuse)
